# baseline (speedup 1.0000x reference)
import functools

import jax
import jax.numpy as jnp
from jax import lax
from jax.experimental import pallas as pl
from jax.experimental.pallas import tpu as pltpu
from jax.experimental.pallas import tpu_sc as plsc

_VOCAB = 64000
_D = 6
_B = 8
_T, _H, _W = 5, 32, 32
_FRAME = _H * _W
_N_KEEP = (_T - 1) * _FRAME
_NC, _NS = 2, 16
_TOK = 512
_CHUNK = 2 * _TOK
_E = _CHUNK * _D
_NW = _E // 2
_OUTW = _D * (_N_KEEP // 128) * (_B // 2) * 128


def _f16_bits(v):
    x = plsc.bitcast(v, jnp.int32)
    am = x & 0x7FFFFFFF
    h = (am >> 13) - 0x1C000 + ((x >> 12) & 1)
    h = jnp.where(am >= 0x38800000, h, 0)
    return ((x >> 16) & 0x8000) | h


def _expand(idx_ref, eidx_ref):
    def body(c, carry):
        vi = idx_ref[pl.ds(c * 16, 16)]
        base = ((vi >> 7) << 10) + (vi & 127)
        off = ((c & 31) << 4) + (c >> 5) * 3072
        for k in range(_D):
            eidx_ref[pl.ds(off + k * 512, 16)] = base + (k << 7)
        return carry
    lax.fori_loop(0, _CHUNK // 16, body, 0)


def _pack(rows_ref, out32_ref):
    def body(n, carry):
        a = rows_ref[pl.ds(n * 16, 16)]
        b = rows_ref[pl.ds(3072 + n * 16, 16)]
        w = _f16_bits(a) | (_f16_bits(b) << 16)
        out32_ref[pl.ds(n * 32, 32)] = plsc.bitcast(w, jnp.float16)
        return carry
    lax.fori_loop(0, _NW // 16, body, 0)


def _sc_body(tgt_idx, pred_idx, cbf, tgt_out, pred_out,
             idx_t, idx_p, eidx_t, eidx_p, rows_t, rows_p, o32_t, o32_p,
             sem_it, sem_ip, sem_t, sem_p, sem_o):
    wid = lax.axis_index("s") * _NC + lax.axis_index("c")
    b2 = wid // 8
    q8 = wid % 8
    j0 = _FRAME // 128 + 4 * q8

    loads = []
    for (src, dst, sem) in ((tgt_idx, idx_t, sem_it), (pred_idx, idx_p, sem_ip)):
        for half in range(2):
            for jj in range(4):
                loads.append(pltpu.async_copy(
                    src.at[j0 + jj, 2 * b2 + half],
                    dst.at[pl.ds(half * 512 + jj * 128, 128)], sem))
    for c in loads[:8]:
        c.wait()
    _expand(idx_t, eidx_t)
    gt = pltpu.async_copy(cbf.at[eidx_t], rows_t, sem_t)
    for c in loads[8:]:
        c.wait()
    _expand(idx_p, eidx_p)
    gp = pltpu.async_copy(cbf.at[eidx_p], rows_p, sem_p)

    stores = []

    def _emit(out32_ref, out_hbm):
        for d in range(_D):
            for jj in range(4):
                dst0 = 2 * (d * 16384 + (4 * q8 + jj) * 512 + b2 * 128)
                stores.append(pltpu.async_copy(
                    out32_ref.at[pl.ds(d * 1024 + jj * 256, 256)],
                    out_hbm.at[pl.ds(dst0, 256)], sem_o))

    gt.wait()
    _pack(rows_t, o32_t)
    _emit(o32_t, tgt_out)
    gp.wait()
    _pack(rows_p, o32_p)
    _emit(o32_p, pred_out)
    for c in stores:
        c.wait()


_sc_gather = functools.partial(
    pl.kernel,
    out_type=(
        jax.ShapeDtypeStruct((2 * _OUTW,), jnp.float16),
        jax.ShapeDtypeStruct((2 * _OUTW,), jnp.float16),
    ),
    mesh=plsc.VectorSubcoreMesh(core_axis_name="c", subcore_axis_name="s"),
    scratch_types=[
        pltpu.VMEM((_CHUNK,), jnp.int32),
        pltpu.VMEM((_CHUNK,), jnp.int32),
        pltpu.VMEM((_E,), jnp.int32),
        pltpu.VMEM((_E,), jnp.int32),
        pltpu.VMEM((_E,), jnp.float32),
        pltpu.VMEM((_E,), jnp.float32),
        pltpu.VMEM((2 * _NW,), jnp.float16),
        pltpu.VMEM((2 * _NW,), jnp.float16),
        pltpu.SemaphoreType.DMA,
        pltpu.SemaphoreType.DMA,
        pltpu.SemaphoreType.DMA,
        pltpu.SemaphoreType.DMA,
        pltpu.SemaphoreType.DMA,
    ],
    compiler_params=pltpu.CompilerParams(use_tc_tiling_on_sc=False,
                                         needs_layout_passes=False),
)(_sc_body)


def _tiled_view(idx):
    return idx.reshape(_B, 40, 128).transpose(1, 0, 2)


def _finish(o):
    h = o.reshape(_D, _N_KEEP // 128, _B // 2, 128, 2)
    return h.transpose(2, 4, 1, 3, 0).reshape(_B, _N_KEEP, _D)


def kernel(target_indices, pred_indices, codebook):
    cbp = jnp.pad(codebook.T, ((0, 2), (0, 0)))
    cbf = cbp.reshape(_B, 500, 128).transpose(1, 0, 2).reshape(500 * 1024)
    o_t, o_p = _sc_gather(_tiled_view(target_indices),
                          _tiled_view(pred_indices), cbf)
    return (_finish(o_p), _finish(o_t))


# speedup vs baseline: 4.7094x; 4.7094x over previous
import functools

import jax
import jax.numpy as jnp
from jax import lax
from jax.experimental import pallas as pl
from jax.experimental.pallas import tpu as pltpu
from jax.experimental.pallas import tpu_sc as plsc

_VOCAB = 64000
_D = 6
_B = 8
_T, _H, _W = 5, 32, 32
_FRAME = _H * _W
_N_KEEP = (_T - 1) * _FRAME
_NC, _NS = 2, 16
_TOK = 512
_CHUNK = 2 * _TOK
_E = _CHUNK * _D
_NW = _E // 2
_OUTW = _D * (_N_KEEP // 128) * (_B // 2) * 128


def _f16_bits(v):
    x = plsc.bitcast(v, jnp.int32)
    am = x & 0x7FFFFFFF
    h = (am >> 13) - 0x1C000 + ((x >> 12) & 1)
    h = jnp.where(am >= 0x38800000, h, 0)
    return ((x >> 16) & 0x8000) | h


def _expand(idx_ref, eidx_ref):
    def body(c, carry):
        vi = idx_ref[pl.ds(c * 16, 16)]
        base = ((vi >> 7) << 10) + (vi & 127)
        off = ((c & 31) << 4) + (c >> 5) * 3072
        for k in range(_D):
            eidx_ref[pl.ds(off + k * 512, 16)] = base + (k << 7)
        return carry
    lax.fori_loop(0, _CHUNK // 16, body, 0)


def _pack(rows_ref, out32_ref):
    def body(n, carry):
        a = rows_ref[pl.ds(n * 16, 16)]
        b = rows_ref[pl.ds(3072 + n * 16, 16)]
        out32_ref[pl.ds(n * 16, 16)] = _f16_bits(a) | (_f16_bits(b) << 16)
        return carry
    lax.fori_loop(0, _NW // 16, body, 0)


def _sc_body(tgt_idx, pred_idx, cbf, tgt_out, pred_out,
             idx_t, idx_p, eidx_t, eidx_p, rows_t, rows_p, o32_t, o32_p,
             sem_it, sem_ip, sem_t, sem_p, sem_o):
    wid = lax.axis_index("s") * _NC + lax.axis_index("c")
    b2 = wid // 8
    q8 = wid % 8
    j0 = _FRAME // 128 + 4 * q8

    loads = []
    for (src, dst, sem) in ((tgt_idx, idx_t, sem_it), (pred_idx, idx_p, sem_ip)):
        for half in range(2):
            for jj in range(4):
                loads.append(pltpu.async_copy(
                    src.at[j0 + jj, 2 * b2 + half],
                    dst.at[pl.ds(half * 512 + jj * 128, 128)], sem))
    for c in loads[:8]:
        c.wait()
    _expand(idx_t, eidx_t)
    gt = pltpu.async_copy(cbf.at[eidx_t], rows_t, sem_t)
    for c in loads[8:]:
        c.wait()
    _expand(idx_p, eidx_p)
    gp = pltpu.async_copy(cbf.at[eidx_p], rows_p, sem_p)

    stores = []

    def _emit(out32_ref, out_hbm):
        for d in range(_D):
            for jj in range(4):
                dst0 = d * 16384 + (4 * q8 + jj) * 512 + b2 * 128
                stores.append(pltpu.async_copy(
                    out32_ref.at[pl.ds(d * 512 + jj * 128, 128)],
                    out_hbm.at[pl.ds(dst0, 128)], sem_o))

    gt.wait()
    _pack(rows_t, o32_t)
    _emit(o32_t, tgt_out)
    gp.wait()
    _pack(rows_p, o32_p)
    _emit(o32_p, pred_out)
    for c in stores:
        c.wait()


_sc_gather = functools.partial(
    pl.kernel,
    out_type=(
        jax.ShapeDtypeStruct((_OUTW,), jnp.int32),
        jax.ShapeDtypeStruct((_OUTW,), jnp.int32),
    ),
    mesh=plsc.VectorSubcoreMesh(core_axis_name="c", subcore_axis_name="s"),
    scratch_types=[
        pltpu.VMEM((_CHUNK,), jnp.int32),
        pltpu.VMEM((_CHUNK,), jnp.int32),
        pltpu.VMEM((_E,), jnp.int32),
        pltpu.VMEM((_E,), jnp.int32),
        pltpu.VMEM((_E,), jnp.float32),
        pltpu.VMEM((_E,), jnp.float32),
        pltpu.VMEM((_NW,), jnp.int32),
        pltpu.VMEM((_NW,), jnp.int32),
        pltpu.SemaphoreType.DMA,
        pltpu.SemaphoreType.DMA,
        pltpu.SemaphoreType.DMA,
        pltpu.SemaphoreType.DMA,
        pltpu.SemaphoreType.DMA,
    ],
    compiler_params=pltpu.CompilerParams(use_tc_tiling_on_sc=False,
                                         needs_layout_passes=False),
)(_sc_body)


def _tiled_view(idx):
    return idx.reshape(_B, 40, 128).transpose(1, 0, 2)


def _finish(out32):
    h = lax.bitcast_convert_type(out32, jnp.float16)
    h = h.reshape(_D, _N_KEEP // 128, _B // 2, 128, 2)
    return h.transpose(2, 4, 1, 3, 0).reshape(_B, _N_KEEP, _D)


def kernel(target_indices, pred_indices, codebook):
    cbp = jnp.pad(codebook.T, ((0, 2), (0, 0)))
    cbf = cbp.reshape(_B, 500, 128).transpose(1, 0, 2).reshape(500 * 1024)
    o_t, o_p = _sc_gather(_tiled_view(target_indices),
                          _tiled_view(pred_indices), cbf)
    return (_finish(o_p), _finish(o_t))
